# Initial kernel scaffold; baseline (speedup 1.0000x reference)
import jax
import jax.numpy as jnp
from jax.experimental import pallas as pl


def kernel(features, Ck):
    raise NotImplementedError("write your pallas kernel here")


# speedup vs baseline: 1.0841x; 1.0841x over previous
import functools

import jax
import jax.numpy as jnp
from jax.experimental import pallas as pl
from jax.experimental.pallas import tpu as pltpu


def _dist_kernel(f_ref, c_ref, o_ref):
    f = f_ref[...]
    c = c_ref[...]
    f2 = jnp.sum(f * f, axis=1, keepdims=True)
    c2 = jnp.sum(c * c, axis=1)[None, :]
    fc = jax.lax.dot_general(
        f, c, (((1,), (1,)), ((), ())),
        preferred_element_type=jnp.float32,
    )
    o_ref[...] = (f2 + c2) - 2.0 * fc


@functools.partial(jax.jit, static_argnames=("bm",))
def _dists(f, c, bm):
    M, D = f.shape
    K = c.shape[0]
    grid = (M // bm,)
    return pl.pallas_call(
        _dist_kernel,
        grid=grid,
        in_specs=[
            pl.BlockSpec((bm, D), lambda i: (i, 0)),
            pl.BlockSpec((K, D), lambda i: (0, 0)),
        ],
        out_specs=pl.BlockSpec((bm, K), lambda i: (i, 0)),
        out_shape=jax.ShapeDtypeStruct((M, K), jnp.float32),
        compiler_params=pltpu.CompilerParams(
            dimension_semantics=("arbitrary",),
        ),
    )(f, c)


def kernel(features, Ck):
    B, S, D = features.shape
    K = Ck.shape[1]
    f = features.reshape(B * S, D)
    c = Ck.reshape(K, D)
    dists = _dists(f, c, bm=512)
    return dists.reshape(B, S, K)
